# Initial kernel scaffold; baseline (speedup 1.0000x reference)
import jax
import jax.numpy as jnp
from jax.experimental import pallas as pl


def kernel(x, edge_index, adj_emb, Wx1, bx1, Wx2, bx2, Wa, ba, Wf1, bf1, Wf2, bf2, Wo, bo):
    raise NotImplementedError("write your pallas kernel here")


# speedup vs baseline: 3.6347x; 3.6347x over previous
import functools

import jax
import jax.numpy as jnp
from jax import lax
from jax.experimental import pallas as pl
from jax.experimental.pallas import tpu as pltpu
from jax.experimental.pallas import tpu_sc as plsc

N = 10000
E = 320000
D = 128
H = 128
C = 64

NC = 2
NS = 16
W = NC * NS
CH = 64
EP = 327680
EPW = EP // W
ITERS = EPW // CH
GRP = 16
NGRP = ITERS // GRP
NP = 10112
RPT = NP // NS
CW = 16


def _sc_segment_sum(src_r, dst_r, adj_emb, z_acc, z_cnt, ones16):
    mesh = plsc.VectorSubcoreMesh(core_axis_name="c", subcore_axis_name="s")

    @functools.partial(
        pl.kernel,
        out_type=[
            jax.ShapeDtypeStruct((NC * NP, H), jnp.float32),
            jax.ShapeDtypeStruct((NC * NP, CW), jnp.float32),
        ],
        mesh=mesh,
        compiler_params=pltpu.CompilerParams(use_tc_tiling_on_sc=False),
        scratch_types=[
            pltpu.VMEM((GRP, CH), jnp.int32),
            pltpu.VMEM((GRP, CH), jnp.int32),
            pltpu.VMEM((CH, H), jnp.float32),
            pltpu.VMEM((CH, CW), jnp.float32),
            pltpu.VMEM_SHARED((NP, H), jnp.float32),
            pltpu.VMEM_SHARED((NP, CW), jnp.float32),
            pltpu.SemaphoreType.DMA,
        ],
    )
    def seg(src_hbm, dst_hbm, adj_hbm, zacc_hbm, zcnt_hbm, ones_hbm,
            psum_hbm, pcnt_hbm,
            src_v, dst_v, rows_v, ones_v, acc_sh, cnt_sh, sem):
        c = lax.axis_index("c")
        s = lax.axis_index("s")
        wid = s * NC + c
        r0 = s * RPT
        pltpu.sync_copy(zacc_hbm.at[pl.ds(r0, RPT)], acc_sh.at[pl.ds(r0, RPT)])
        pltpu.sync_copy(zcnt_hbm.at[pl.ds(r0, RPT)], cnt_sh.at[pl.ds(r0, RPT)])
        pltpu.sync_copy(ones_hbm, ones_v)
        plsc.subcore_barrier()

        def group(g, carry):
            pltpu.sync_copy(src_hbm.at[wid, pl.ds(g * GRP, GRP)], src_v)
            pltpu.sync_copy(dst_hbm.at[wid, pl.ds(g * GRP, GRP)], dst_v)

            def step(k, carry2):
                pltpu.async_copy(adj_hbm.at[dst_v.at[k]], rows_v, sem).wait()
                pltpu.sync_copy(rows_v, acc_sh.at[src_v.at[k]], add=True)
                pltpu.sync_copy(ones_v, cnt_sh.at[src_v.at[k]], add=True)
                return carry2

            return lax.fori_loop(0, GRP, step, carry)

        lax.fori_loop(0, NGRP, group, 0)
        plsc.subcore_barrier()
        o0 = c * NP + r0
        pltpu.sync_copy(acc_sh.at[pl.ds(r0, RPT)], psum_hbm.at[pl.ds(o0, RPT)])
        pltpu.sync_copy(cnt_sh.at[pl.ds(r0, RPT)], pcnt_hbm.at[pl.ds(o0, RPT)])

    return seg(src_r, dst_r, adj_emb, z_acc, z_cnt, ones16)


def _tc_body(x_ref, s0_ref, s1_ref, c0_ref, c1_ref,
             wx1, bx1, wx2, bx2, wa, ba, wf1, bf1, wf2, bf2, wo, bo,
             out_ref):
    dot = functools.partial(jnp.dot, preferred_element_type=jnp.float32)
    x = x_ref[...]
    hx = dot(jnp.maximum(dot(x, wx1[...]) + bx1[...], 0.0), wx2[...]) + bx2[...]
    cnt = jnp.maximum(c0_ref[:, 0:1] + c1_ref[:, 0:1], 1.0)
    mean = (s0_ref[...] + s1_ref[...]) / cnt
    wfa = wf1[0:H] + wf1[2 * H:3 * H]
    wfx = wf1[H:2 * H] + wf1[2 * H:3 * H]
    wa2 = dot(wa[...], wfa)
    ba2 = dot(ba[...], wfa)
    t = jnp.maximum(dot(mean, wa2) + dot(hx, wfx) + ba2 + bf1[...], 0.0)
    z = jnp.maximum(dot(t, wf2[...]) + bf2[...], 0.0)
    out_ref[...] = dot(z, wo[...]) + bo[...]


def _tc_dense(x, s0, s1, c0, c1, Wx1, bx1, Wx2, bx2, Wa, ba, Wf1, bf1, Wf2, bf2, Wo, bo):
    B = 512
    row = lambda i: (i, 0)
    full = lambda i: (0, 0)
    return pl.pallas_call(
        _tc_body,
        grid=(20,),
        in_specs=[
            pl.BlockSpec((B, D), row),
            pl.BlockSpec((B, H), row),
            pl.BlockSpec((B, H), row),
            pl.BlockSpec((B, CW), row),
            pl.BlockSpec((B, CW), row),
            pl.BlockSpec((D, H), full),
            pl.BlockSpec((1, H), full),
            pl.BlockSpec((H, H), full),
            pl.BlockSpec((1, H), full),
            pl.BlockSpec((H, H), full),
            pl.BlockSpec((1, H), full),
            pl.BlockSpec((3 * H, H), full),
            pl.BlockSpec((1, H), full),
            pl.BlockSpec((H, H), full),
            pl.BlockSpec((1, H), full),
            pl.BlockSpec((H, C), full),
            pl.BlockSpec((1, C), full),
        ],
        out_specs=pl.BlockSpec((B, C), row),
        out_shape=jax.ShapeDtypeStruct((N, C), jnp.float32),
    )(x, s0, s1, c0, c1, Wx1, bx1.reshape(1, H), Wx2, bx2.reshape(1, H),
      Wa, ba.reshape(1, H), Wf1, bf1.reshape(1, H), Wf2, bf2.reshape(1, H),
      Wo, bo.reshape(1, C))


def kernel(x, edge_index, adj_emb, Wx1, bx1, Wx2, bx2, Wa, ba, Wf1, bf1, Wf2, bf2, Wo, bo):
    pad = EP - E
    src_p = jnp.concatenate([edge_index[0], jnp.full((pad,), N, jnp.int32)])
    dst_p = jnp.concatenate([edge_index[1], jnp.zeros((pad,), jnp.int32)])
    src_r = src_p.reshape(W, ITERS, CH)
    dst_r = dst_p.reshape(W, ITERS, CH)
    z_acc = jnp.zeros((NP, H), jnp.float32)
    z_cnt = jnp.zeros((NP, CW), jnp.float32)
    ones16 = jnp.ones((CH, CW), jnp.float32)
    psum, pcnt = _sc_segment_sum(src_r, dst_r, adj_emb, z_acc, z_cnt, ones16)
    s0, s1 = psum[:NP], psum[NP:]
    c0, c1 = pcnt[:NP], pcnt[NP:]
    return _tc_dense(x, s0, s1, c0, c1, Wx1, bx1, Wx2, bx2,
                     Wa, ba, Wf1, bf1, Wf2, bf2, Wo, bo)
